# Initial kernel scaffold; baseline (speedup 1.0000x reference)
import jax
import jax.numpy as jnp
from jax.experimental import pallas as pl


def kernel(query, memory, adj_indices, W1p, b1p, g1p, be1p, W2p, b2p, g2p, be2p, W3p, b3p, W1m, b1m, g1m, be1m, W2m, b2m, g2m, be2m, W3m, b3m):
    raise NotImplementedError("write your pallas kernel here")


# speedup vs baseline: 2.7391x; 2.7391x over previous
import functools

import jax
import jax.numpy as jnp
from jax import lax
from jax.experimental import pallas as pl
from jax.experimental.pallas import tpu as pltpu
from jax.experimental.pallas import tpu_sc as plsc


def _ln(x, g, b):
    mu = jnp.mean(x, axis=-1, keepdims=True)
    var = jnp.mean((x - mu) * (x - mu), axis=-1, keepdims=True)
    return (x - mu) / jnp.sqrt(var + 1e-3) * g + b


def _pre_body(q_ref, m_ref, w1q_ref, w1k_ref, b1p_ref,
              w1m_ref, b1m_ref, g1m_ref, be1m_ref,
              w2m_ref, b2m_ref, g2m_ref, be2m_ref,
              w3m_ref, b3m_ref,
              a_ref, b_ref, k_ref):
    q = q_ref[...]
    m = m_ref[...]
    a_ref[...] = jnp.dot(q, w1q_ref[...],
                         preferred_element_type=jnp.float32) + b1p_ref[...]
    b_ref[...] = jnp.dot(m, w1k_ref[...],
                         preferred_element_type=jnp.float32)
    h = jnp.dot(m, w1m_ref[...], preferred_element_type=jnp.float32)
    h = jnp.maximum(_ln(h + b1m_ref[...], g1m_ref[...], be1m_ref[...]), 0.0)
    h = jnp.dot(h, w2m_ref[...], preferred_element_type=jnp.float32)
    h = jnp.maximum(_ln(h + b2m_ref[...], g2m_ref[...], be2m_ref[...]), 0.0)
    k_ref[...] = jnp.dot(h, w3m_ref[...],
                         preferred_element_type=jnp.float32) + b3m_ref[...]


def _precompute(query, memory, w1q, w1k, b1p,
                w1m, b1m, g1m, be1m, w2m, b2m, g2m, be2m, w3m, b3m):
    n, _ = query.shape
    h_dim = w1q.shape[1]
    out_shape = (jax.ShapeDtypeStruct((n, h_dim), jnp.float32),
                 jax.ShapeDtypeStruct((n, h_dim), jnp.float32),
                 jax.ShapeDtypeStruct((n, h_dim), jnp.float32))
    return pl.pallas_call(_pre_body, out_shape=out_shape)(
        query, memory, w1q, w1k, b1p,
        w1m, b1m, g1m, be1m, w2m, b2m, g2m, be2m, w3m, b3m)


def _mlp_body(x_ref, g1_ref, be1_ref, w2_ref, b2_ref, g2_ref, be2_ref,
              w3_ref, b3_ref, h_ref):
    x = x_ref[0]
    h1 = jnp.maximum(_ln(x, g1_ref[...], be1_ref[...]), 0.0)
    h2 = jnp.dot(h1, w2_ref[...], preferred_element_type=jnp.float32) + b2_ref[...]
    h2 = jnp.maximum(_ln(h2, g2_ref[...], be2_ref[...]), 0.0)
    t = jnp.sum(h2 * w3_ref[...], axis=-1) + b3_ref[0, 0]
    h_ref[0, 0] = jnp.tanh(t)


def _edge_mlp(x3, g1p, be1p, w2p, b2p, g2p, be2p, w3row, b3):
    nb, be, h_dim = x3.shape
    wspec = pl.BlockSpec((1, h_dim), lambda i: (0, 0))
    mspec = pl.BlockSpec((h_dim, h_dim), lambda i: (0, 0))
    return pl.pallas_call(
        _mlp_body,
        grid=(nb,),
        in_specs=[pl.BlockSpec((1, be, h_dim), lambda i: (i, 0, 0)),
                  wspec, wspec, mspec, wspec, wspec, wspec, wspec,
                  pl.BlockSpec((1, 1), lambda i: (0, 0))],
        out_specs=pl.BlockSpec((1, 1, be), lambda i: (i, 0, 0)),
        out_shape=jax.ShapeDtypeStruct((nb, 1, be), jnp.float32),
    )(x3, g1p, be1p, w2p, b2p, g2p, be2p, w3row, b3)


def _add_body(a_ref, b_ref, o_ref):
    o_ref[...] = a_ref[...] + b_ref[...]


def _combine(p0, p1):
    return pl.pallas_call(
        _add_body, out_shape=jax.ShapeDtypeStruct(p0.shape, jnp.float32))(p0, p1)


def _make_sc_gather(e_total, h_dim, nc, ns, c):
    nw = nc * ns
    ew = e_total // nw
    nch = ew // c
    mesh = plsc.VectorSubcoreMesh(core_axis_name="c", subcore_axis_name="s")

    @functools.partial(
        pl.kernel, mesh=mesh,
        compiler_params=pltpu.CompilerParams(use_tc_tiling_on_sc=False,
                                             needs_layout_passes=False),
        out_type=jax.ShapeDtypeStruct((e_total, h_dim), jnp.float32),
        scratch_types=[pltpu.VMEM((c,), jnp.int32),
                       pltpu.VMEM((c,), jnp.int32),
                       pltpu.VMEM((c, h_dim), jnp.float32),
                       pltpu.VMEM((c, h_dim), jnp.float32),
                       pltpu.SemaphoreType.DMA],
    )
    def gk(a_hbm, b_hbm, src_hbm, dst_hbm, x_hbm, sv, dv, ra, rb, sem):
        wid = lax.axis_index("s") * nc + lax.axis_index("c")

        def chunk(ci, carry):
            base = wid * ew + ci * c
            pltpu.sync_copy(src_hbm.at[pl.ds(base, c)], sv)
            pltpu.sync_copy(dst_hbm.at[pl.ds(base, c)], dv)
            pltpu.async_copy(a_hbm.at[sv], ra, sem).wait()
            pltpu.async_copy(b_hbm.at[dv], rb, sem).wait()

            def row(i, rcarry):
                for j in range(h_dim // 16):
                    s = pl.ds(j * 16, 16)
                    ra[i, s] = ra[i, s] + rb[i, s]
                return rcarry

            lax.fori_loop(0, c, row, 0)
            pltpu.sync_copy(ra, x_hbm.at[pl.ds(base, c)])
            return carry

        lax.fori_loop(0, nch, chunk, 0)

    return gk


def _make_sc_scatter(n, e_total, h_dim, nc, ns, c):
    nw = nc * ns
    ew = e_total // nw
    nch = ew // c
    mesh = plsc.VectorSubcoreMesh(core_axis_name="c", subcore_axis_name="s")

    @functools.partial(
        pl.kernel, mesh=mesh,
        compiler_params=pltpu.CompilerParams(use_tc_tiling_on_sc=False,
                                             needs_layout_passes=False),
        out_type=jax.ShapeDtypeStruct((nc, n, h_dim), jnp.float32),
        scratch_types=[pltpu.VMEM((c,), jnp.int32),
                       pltpu.VMEM((c,), jnp.int32),
                       pltpu.VMEM((c,), jnp.float32),
                       pltpu.VMEM((c, h_dim), jnp.float32),
                       pltpu.VMEM_SHARED((n, h_dim), jnp.float32),
                       pltpu.SemaphoreType.DMA],
    )
    def sk(k_hbm, h_hbm, src_hbm, dst_hbm, z_hbm, out_hbm,
           sv, dv, hv, rows, acc, sem):
        cid = lax.axis_index("c")
        sid = lax.axis_index("s")
        wid = sid * nc + cid

        @pl.when(sid == 0)
        def _():
            pltpu.sync_copy(z_hbm, acc)

        plsc.subcore_barrier()

        def chunk(ci, carry):
            base = wid * ew + ci * c
            pltpu.sync_copy(src_hbm.at[pl.ds(base, c)], sv)
            pltpu.sync_copy(dst_hbm.at[pl.ds(base, c)], dv)
            pltpu.sync_copy(h_hbm.at[pl.ds(base, c)], hv)
            pltpu.async_copy(k_hbm.at[dv], rows, sem).wait()

            def row(i, rcarry):
                hb = plsc.load_gather(hv, [jnp.full((16,), i, jnp.int32)])
                for j in range(h_dim // 16):
                    s = pl.ds(j * 16, 16)
                    rows[i, s] = rows[i, s] * hb
                return rcarry

            lax.fori_loop(0, c, row, 0)
            pltpu.sync_copy(rows, acc.at[sv], add=True)
            return carry

        lax.fori_loop(0, nch, chunk, 0)
        plsc.subcore_barrier()

        @pl.when(sid == 0)
        def _():
            pltpu.sync_copy(acc, out_hbm.at[cid])

    return sk


def kernel(query, memory, adj_indices, W1p, b1p, g1p, be1p, W2p, b2p, g2p,
           be2p, W3p, b3p, W1m, b1m, g1m, be1m, W2m, b2m, g2m, be2m, W3m,
           b3m):
    n, d = query.shape
    e_total = adj_indices.shape[0]
    h_dim = W1p.shape[1]

    info = plsc.get_sparse_core_info()
    nc, ns = info.num_cores, info.num_subcores
    c = 80

    src = adj_indices[:, 0]
    dst = adj_indices[:, 1]

    a_tab, b_tab, k_tab = _precompute(
        query, memory, W1p[:d], W1p[d:], b1p.reshape(1, h_dim),
        W1m, b1m.reshape(1, h_dim), g1m.reshape(1, h_dim),
        be1m.reshape(1, h_dim), W2m, b2m.reshape(1, h_dim),
        g2m.reshape(1, h_dim), be2m.reshape(1, h_dim), W3m,
        b3m.reshape(1, h_dim))

    x = _make_sc_gather(e_total, h_dim, nc, ns, c)(a_tab, b_tab, src, dst)

    be = 2560
    nb = e_total // be
    h3 = _edge_mlp(x.reshape(nb, be, h_dim),
                   g1p.reshape(1, h_dim), be1p.reshape(1, h_dim),
                   W2p, b2p.reshape(1, h_dim),
                   g2p.reshape(1, h_dim), be2p.reshape(1, h_dim),
                   W3p.reshape(1, h_dim), b3p.reshape(1, 1))
    h_edges = h3.reshape(e_total)

    zeros = jnp.zeros((n, h_dim), jnp.float32)
    partials = _make_sc_scatter(n, e_total, h_dim, nc, ns, c)(
        k_tab, h_edges, src, dst, zeros)

    return _combine(partials[0], partials[1])


# speedup vs baseline: 3.0588x; 1.1167x over previous
import functools

import jax
import jax.numpy as jnp
from jax import lax
from jax.experimental import pallas as pl
from jax.experimental.pallas import tpu as pltpu
from jax.experimental.pallas import tpu_sc as plsc


def _ln(x, g, b):
    mu = jnp.mean(x, axis=-1, keepdims=True)
    var = jnp.mean((x - mu) * (x - mu), axis=-1, keepdims=True)
    return (x - mu) / jnp.sqrt(var + 1e-3) * g + b


def _pre_body(q_ref, m_ref, w1q_ref, w1k_ref, b1p_ref,
              w1m_ref, b1m_ref, g1m_ref, be1m_ref,
              w2m_ref, b2m_ref, g2m_ref, be2m_ref,
              w3m_ref, b3m_ref,
              a_ref, b_ref, k_ref):
    q = q_ref[...]
    m = m_ref[...]
    a_ref[...] = jnp.dot(q, w1q_ref[...],
                         preferred_element_type=jnp.float32) + b1p_ref[...]
    b_ref[...] = jnp.dot(m, w1k_ref[...],
                         preferred_element_type=jnp.float32)
    h = jnp.dot(m, w1m_ref[...], preferred_element_type=jnp.float32)
    h = jnp.maximum(_ln(h + b1m_ref[...], g1m_ref[...], be1m_ref[...]), 0.0)
    h = jnp.dot(h, w2m_ref[...], preferred_element_type=jnp.float32)
    h = jnp.maximum(_ln(h + b2m_ref[...], g2m_ref[...], be2m_ref[...]), 0.0)
    k_ref[...] = jnp.dot(h, w3m_ref[...],
                         preferred_element_type=jnp.float32) + b3m_ref[...]


def _precompute(query, memory, w1q, w1k, b1p,
                w1m, b1m, g1m, be1m, w2m, b2m, g2m, be2m, w3m, b3m):
    n, _ = query.shape
    h_dim = w1q.shape[1]
    out_shape = (jax.ShapeDtypeStruct((n, h_dim), jnp.float32),
                 jax.ShapeDtypeStruct((n, h_dim), jnp.float32),
                 jax.ShapeDtypeStruct((n, h_dim), jnp.float32))
    return pl.pallas_call(_pre_body, out_shape=out_shape)(
        query, memory, w1q, w1k, b1p,
        w1m, b1m, g1m, be1m, w2m, b2m, g2m, be2m, w3m, b3m)


def _mlp_body(x_ref, g1_ref, be1_ref, w2_ref, b2_ref, g2_ref, be2_ref,
              w3_ref, b3_ref, h_ref):
    x = x_ref[0]
    h1 = jnp.maximum(_ln(x, g1_ref[...], be1_ref[...]), 0.0)
    h2 = jnp.dot(h1, w2_ref[...], preferred_element_type=jnp.float32) + b2_ref[...]
    h2 = jnp.maximum(_ln(h2, g2_ref[...], be2_ref[...]), 0.0)
    t = jnp.sum(h2 * w3_ref[...], axis=-1) + b3_ref[0, 0]
    h_ref[0, 0] = jnp.tanh(t)


def _edge_mlp(x3, g1p, be1p, w2p, b2p, g2p, be2p, w3row, b3):
    nb, be, h_dim = x3.shape
    wspec = pl.BlockSpec((1, h_dim), lambda i: (0, 0))
    mspec = pl.BlockSpec((h_dim, h_dim), lambda i: (0, 0))
    return pl.pallas_call(
        _mlp_body,
        grid=(nb,),
        in_specs=[pl.BlockSpec((1, be, h_dim), lambda i: (i, 0, 0)),
                  wspec, wspec, mspec, wspec, wspec, wspec, wspec,
                  pl.BlockSpec((1, 1), lambda i: (0, 0))],
        out_specs=pl.BlockSpec((1, 1, be), lambda i: (i, 0, 0)),
        out_shape=jax.ShapeDtypeStruct((nb, 1, be), jnp.float32),
    )(x3, g1p, be1p, w2p, b2p, g2p, be2p, w3row, b3)


def _add_body(a_ref, b_ref, o_ref):
    o_ref[...] = a_ref[...] + b_ref[...]


def _combine(p0, p1):
    return pl.pallas_call(
        _add_body, out_shape=jax.ShapeDtypeStruct(p0.shape, jnp.float32))(p0, p1)


def _make_sc_gather(e_total, h_dim, nc, ns, c):
    nw = nc * ns
    ew = e_total // nw
    nch = ew // c
    mesh = plsc.VectorSubcoreMesh(core_axis_name="c", subcore_axis_name="s")

    @functools.partial(
        pl.kernel, mesh=mesh,
        compiler_params=pltpu.CompilerParams(use_tc_tiling_on_sc=False,
                                             needs_layout_passes=False),
        out_type=jax.ShapeDtypeStruct((e_total, h_dim), jnp.float32),
        scratch_types=[pltpu.VMEM((c,), jnp.int32),
                       pltpu.VMEM((c,), jnp.int32),
                       pltpu.VMEM((c, h_dim), jnp.float32),
                       pltpu.VMEM((c, h_dim), jnp.float32),
                       pltpu.SemaphoreType.DMA],
    )
    def gk(a_hbm, b_hbm, src_hbm, dst_hbm, x_hbm, sv, dv, ra, rb, sem):
        wid = lax.axis_index("s") * nc + lax.axis_index("c")

        def chunk(ci, carry):
            base = wid * ew + ci * c
            pltpu.sync_copy(src_hbm.at[pl.ds(base, c)], sv)
            pltpu.sync_copy(dst_hbm.at[pl.ds(base, c)], dv)
            pltpu.async_copy(a_hbm.at[sv], ra, sem).wait()
            pltpu.async_copy(b_hbm.at[dv], rb, sem).wait()

            def row(i, rcarry):
                for j in range(h_dim // 16):
                    s = pl.ds(j * 16, 16)
                    ra[i, s] = ra[i, s] + rb[i, s]
                return rcarry

            lax.fori_loop(0, c, row, 0)
            pltpu.sync_copy(ra, x_hbm.at[pl.ds(base, c)])
            return carry

        lax.fori_loop(0, nch, chunk, 0)

    return gk


def _make_sc_scatter(n, e_total, h_dim, nc, ns, c):
    nw = nc * ns
    ew = e_total // nw
    nch = ew // c
    mesh = plsc.VectorSubcoreMesh(core_axis_name="c", subcore_axis_name="s")

    @functools.partial(
        pl.kernel, mesh=mesh,
        compiler_params=pltpu.CompilerParams(use_tc_tiling_on_sc=False,
                                             needs_layout_passes=False),
        out_type=jax.ShapeDtypeStruct((nc, n, h_dim), jnp.float32),
        scratch_types=[pltpu.VMEM((nch, c), jnp.int32),
                       pltpu.VMEM((2, c, h_dim), jnp.float32),
                       pltpu.VMEM((c,), jnp.int32),
                       pltpu.VMEM((c,), jnp.float32),
                       pltpu.VMEM_SHARED((n, h_dim), jnp.float32),
                       pltpu.SemaphoreType.DMA,
                       pltpu.SemaphoreType.DMA],
    )
    def sk(k_hbm, h_hbm, src_hbm, dst_hbm, z_hbm, out_hbm,
           didx, rows, sv, hv, acc, gsem0, gsem1):
        cid = lax.axis_index("c")
        sid = lax.axis_index("s")
        wid = sid * nc + cid

        @pl.when(sid == 0)
        def _():
            pltpu.sync_copy(z_hbm, acc)

        pltpu.sync_copy(dst_hbm.at[wid], didx)
        plsc.subcore_barrier()
        gsems = (gsem0, gsem1)

        def scale_scatter(ci, b, hdl):
            base = wid * ew + ci * c
            pltpu.sync_copy(src_hbm.at[pl.ds(base, c)], sv)
            pltpu.sync_copy(h_hbm.at[pl.ds(base, c)], hv)
            hdl.wait()

            def row(i, rcarry):
                hb = plsc.load_gather(hv, [jnp.full((16,), i, jnp.int32)])
                for j in range(h_dim // 16):
                    s = pl.ds(j * 16, 16)
                    rows[b, i, s] = rows[b, i, s] * hb
                return rcarry

            lax.fori_loop(0, c, row, 0)
            pltpu.sync_copy(rows.at[b], acc.at[sv], add=True)

        def chunk(ci, carry):
            hdl = pltpu.async_copy(k_hbm.at[didx.at[ci]], rows.at[0],
                                   gsems[0])
            scale_scatter(ci, 0, hdl)
            return carry

        lax.fori_loop(0, nch, chunk, 0)
        plsc.subcore_barrier()

        @pl.when(sid == 0)
        def _():
            pltpu.sync_copy(acc, out_hbm.at[cid])

    return sk


def kernel(query, memory, adj_indices, W1p, b1p, g1p, be1p, W2p, b2p, g2p,
           be2p, W3p, b3p, W1m, b1m, g1m, be1m, W2m, b2m, g2m, be2m, W3m,
           b3m):
    n, d = query.shape
    e_total = adj_indices.shape[0]
    h_dim = W1p.shape[1]

    info = plsc.get_sparse_core_info()
    nc, ns = info.num_cores, info.num_subcores
    c = 80

    nw = nc * ns
    nch = e_total // (nw * c)
    src = adj_indices[:, 0]
    dst = adj_indices[:, 1]
    src3 = src.reshape(nw, nch, c)
    dst3 = dst.reshape(nw, nch, c)

    a_tab, b_tab, k_tab = _precompute(
        query, memory, W1p[:d], W1p[d:], b1p.reshape(1, h_dim),
        W1m, b1m.reshape(1, h_dim), g1m.reshape(1, h_dim),
        be1m.reshape(1, h_dim), W2m, b2m.reshape(1, h_dim),
        g2m.reshape(1, h_dim), be2m.reshape(1, h_dim), W3m,
        b3m.reshape(1, h_dim))

    x = _make_sc_gather(e_total, h_dim, nc, ns, c)(a_tab, b_tab, src, dst)

    be = 2560
    nb = e_total // be
    h3 = _edge_mlp(x.reshape(nb, be, h_dim),
                   g1p.reshape(1, h_dim), be1p.reshape(1, h_dim),
                   W2p, b2p.reshape(1, h_dim),
                   g2p.reshape(1, h_dim), be2p.reshape(1, h_dim),
                   W3p.reshape(1, h_dim), b3p.reshape(1, 1))
    h_edges = h3.reshape(e_total)

    zeros = jnp.zeros((n, h_dim), jnp.float32)
    partials = _make_sc_scatter(n, e_total, h_dim, nc, ns, c)(
        k_tab, h_edges, src, dst3, zeros)

    return _combine(partials[0], partials[1])


# speedup vs baseline: 3.3288x; 1.0883x over previous
import functools

import jax
import jax.numpy as jnp
from jax import lax
from jax.experimental import pallas as pl
from jax.experimental.pallas import tpu as pltpu
from jax.experimental.pallas import tpu_sc as plsc


def _ln(x, g, b):
    mu = jnp.mean(x, axis=-1, keepdims=True)
    var = jnp.mean((x - mu) * (x - mu), axis=-1, keepdims=True)
    return (x - mu) / jnp.sqrt(var + 1e-3) * g + b


def _pre_body(q_ref, m_ref, w1q_ref, w1k_ref, b1p_ref,
              w1m_ref, b1m_ref, g1m_ref, be1m_ref,
              w2m_ref, b2m_ref, g2m_ref, be2m_ref,
              w3m_ref, b3m_ref,
              a_ref, b_ref, k_ref):
    q = q_ref[...]
    m = m_ref[...]
    a_ref[...] = jnp.dot(q, w1q_ref[...],
                         preferred_element_type=jnp.float32) + b1p_ref[...]
    b_ref[...] = jnp.dot(m, w1k_ref[...],
                         preferred_element_type=jnp.float32)
    h = jnp.dot(m, w1m_ref[...], preferred_element_type=jnp.float32)
    h = jnp.maximum(_ln(h + b1m_ref[...], g1m_ref[...], be1m_ref[...]), 0.0)
    h = jnp.dot(h, w2m_ref[...], preferred_element_type=jnp.float32)
    h = jnp.maximum(_ln(h + b2m_ref[...], g2m_ref[...], be2m_ref[...]), 0.0)
    k_ref[...] = jnp.dot(h, w3m_ref[...],
                         preferred_element_type=jnp.float32) + b3m_ref[...]


def _precompute(query, memory, w1q, w1k, b1p,
                w1m, b1m, g1m, be1m, w2m, b2m, g2m, be2m, w3m, b3m):
    n, _ = query.shape
    h_dim = w1q.shape[1]
    out_shape = (jax.ShapeDtypeStruct((n, h_dim), jnp.float32),
                 jax.ShapeDtypeStruct((n, h_dim), jnp.float32),
                 jax.ShapeDtypeStruct((n, h_dim), jnp.float32))
    return pl.pallas_call(_pre_body, out_shape=out_shape)(
        query, memory, w1q, w1k, b1p,
        w1m, b1m, g1m, be1m, w2m, b2m, g2m, be2m, w3m, b3m)


def _mlp_body(x_ref, g1_ref, be1_ref, w2_ref, b2_ref, g2_ref, be2_ref,
              w3_ref, b3_ref, h_ref):
    x = x_ref[0]
    h1 = jnp.maximum(_ln(x, g1_ref[...], be1_ref[...]), 0.0)
    h2 = jnp.dot(h1, w2_ref[...], preferred_element_type=jnp.float32) + b2_ref[...]
    h2 = jnp.maximum(_ln(h2, g2_ref[...], be2_ref[...]), 0.0)
    t = jnp.sum(h2 * w3_ref[...], axis=-1) + b3_ref[0, 0]
    h_ref[0, 0] = jnp.tanh(t)


def _edge_mlp(x3, g1p, be1p, w2p, b2p, g2p, be2p, w3row, b3):
    nb, be, h_dim = x3.shape
    wspec = pl.BlockSpec((1, h_dim), lambda i: (0, 0))
    mspec = pl.BlockSpec((h_dim, h_dim), lambda i: (0, 0))
    return pl.pallas_call(
        _mlp_body,
        grid=(nb,),
        in_specs=[pl.BlockSpec((1, be, h_dim), lambda i: (i, 0, 0)),
                  wspec, wspec, mspec, wspec, wspec, wspec, wspec,
                  pl.BlockSpec((1, 1), lambda i: (0, 0))],
        out_specs=pl.BlockSpec((1, 1, be), lambda i: (i, 0, 0)),
        out_shape=jax.ShapeDtypeStruct((nb, 1, be), jnp.float32),
    )(x3, g1p, be1p, w2p, b2p, g2p, be2p, w3row, b3)


def _add_body(a_ref, b_ref, o_ref):
    o_ref[...] = a_ref[...] + b_ref[...]


def _combine(p0, p1):
    return pl.pallas_call(
        _add_body, out_shape=jax.ShapeDtypeStruct(p0.shape, jnp.float32))(p0, p1)


def _make_sc_gather(e_total, h_dim, nc, ns, c):
    nw = nc * ns
    ew = e_total // nw
    nch = ew // c
    mesh = plsc.VectorSubcoreMesh(core_axis_name="c", subcore_axis_name="s")

    @functools.partial(
        pl.kernel, mesh=mesh,
        compiler_params=pltpu.CompilerParams(use_tc_tiling_on_sc=False,
                                             needs_layout_passes=False),
        out_type=jax.ShapeDtypeStruct((e_total, h_dim), jnp.float32),
        scratch_types=[pltpu.VMEM((nch, c), jnp.int32),
                       pltpu.VMEM((nch, c), jnp.int32),
                       pltpu.VMEM((c, h_dim), jnp.float32),
                       pltpu.VMEM((c, h_dim), jnp.float32),
                       pltpu.SemaphoreType.DMA],
    )
    def gk(a_hbm, b_hbm, src_hbm, dst_hbm, x_hbm, sidx, didx, ra, rb, sem):
        wid = lax.axis_index("s") * nc + lax.axis_index("c")
        pltpu.sync_copy(src_hbm.at[wid], sidx)
        pltpu.sync_copy(dst_hbm.at[wid], didx)

        def chunk(ci, carry):
            base = wid * ew + ci * c
            pltpu.async_copy(a_hbm.at[sidx.at[ci]], ra, sem).wait()
            pltpu.async_copy(b_hbm.at[didx.at[ci]], rb, sem).wait()

            def row(i, rcarry):
                for j in range(h_dim // 16):
                    s = pl.ds(j * 16, 16)
                    ra[i, s] = ra[i, s] + rb[i, s]
                return rcarry

            lax.fori_loop(0, c, row, 0)
            pltpu.sync_copy(ra, x_hbm.at[pl.ds(base, c)])
            return carry

        lax.fori_loop(0, nch, chunk, 0)

    return gk


def _make_sc_scatter(n, e_total, h_dim, nc, ns, c):
    nw = nc * ns
    ew = e_total // nw
    nch = ew // c
    mesh = plsc.VectorSubcoreMesh(core_axis_name="c", subcore_axis_name="s")

    @functools.partial(
        pl.kernel, mesh=mesh,
        compiler_params=pltpu.CompilerParams(use_tc_tiling_on_sc=False,
                                             needs_layout_passes=False),
        out_type=jax.ShapeDtypeStruct((nc, n, h_dim), jnp.float32),
        scratch_types=[pltpu.VMEM((nch, c), jnp.int32),
                       pltpu.VMEM((2, c, h_dim), jnp.float32),
                       pltpu.VMEM((c,), jnp.int32),
                       pltpu.VMEM((c,), jnp.float32),
                       pltpu.VMEM_SHARED((n, h_dim), jnp.float32),
                       pltpu.SemaphoreType.DMA,
                       pltpu.SemaphoreType.DMA],
    )
    def sk(k_hbm, h_hbm, src_hbm, dst_hbm, z_hbm, out_hbm,
           didx, rows, sv, hv, acc, gsem0, gsem1):
        cid = lax.axis_index("c")
        sid = lax.axis_index("s")
        wid = sid * nc + cid

        @pl.when(sid == 0)
        def _():
            pltpu.sync_copy(z_hbm, acc)

        pltpu.sync_copy(dst_hbm.at[wid], didx)
        plsc.subcore_barrier()
        gsems = (gsem0, gsem1)

        def scale_scatter(ci, b, hdl):
            base = wid * ew + ci * c
            pltpu.sync_copy(src_hbm.at[pl.ds(base, c)], sv)
            pltpu.sync_copy(h_hbm.at[pl.ds(base, c)], hv)
            hdl.wait()

            def row(i, rcarry):
                hb = plsc.load_gather(hv, [jnp.full((16,), i, jnp.int32)])
                for j in range(h_dim // 16):
                    s = pl.ds(j * 16, 16)
                    rows[b, i, s] = rows[b, i, s] * hb
                return rcarry

            lax.fori_loop(0, c, row, 0)
            pltpu.sync_copy(rows.at[b], acc.at[sv], add=True)

        def chunk(ci, carry):
            hdl = pltpu.async_copy(k_hbm.at[didx.at[ci]], rows.at[0],
                                   gsems[0])
            scale_scatter(ci, 0, hdl)
            return carry

        lax.fori_loop(0, nch, chunk, 0)
        plsc.subcore_barrier()

        @pl.when(sid == 0)
        def _():
            pltpu.sync_copy(acc, out_hbm.at[cid])

    return sk


def kernel(query, memory, adj_indices, W1p, b1p, g1p, be1p, W2p, b2p, g2p,
           be2p, W3p, b3p, W1m, b1m, g1m, be1m, W2m, b2m, g2m, be2m, W3m,
           b3m):
    n, d = query.shape
    e_total = adj_indices.shape[0]
    h_dim = W1p.shape[1]

    info = plsc.get_sparse_core_info()
    nc, ns = info.num_cores, info.num_subcores
    c = 80

    nw = nc * ns
    nch = e_total // (nw * c)
    src = adj_indices[:, 0]
    dst = adj_indices[:, 1]
    src3 = src.reshape(nw, nch, c)
    dst3 = dst.reshape(nw, nch, c)

    a_tab, b_tab, k_tab = _precompute(
        query, memory, W1p[:d], W1p[d:], b1p.reshape(1, h_dim),
        W1m, b1m.reshape(1, h_dim), g1m.reshape(1, h_dim),
        be1m.reshape(1, h_dim), W2m, b2m.reshape(1, h_dim),
        g2m.reshape(1, h_dim), be2m.reshape(1, h_dim), W3m,
        b3m.reshape(1, h_dim))

    x = _make_sc_gather(e_total, h_dim, nc, ns, c)(a_tab, b_tab, src3, dst3)

    be = 2560
    nb = e_total // be
    h3 = _edge_mlp(x.reshape(nb, be, h_dim),
                   g1p.reshape(1, h_dim), be1p.reshape(1, h_dim),
                   W2p, b2p.reshape(1, h_dim),
                   g2p.reshape(1, h_dim), be2p.reshape(1, h_dim),
                   W3p.reshape(1, h_dim), b3p.reshape(1, 1))
    h_edges = h3.reshape(e_total)

    zeros = jnp.zeros((n, h_dim), jnp.float32)
    partials = _make_sc_scatter(n, e_total, h_dim, nc, ns, c)(
        k_tab, h_edges, src, dst3, zeros)

    return _combine(partials[0], partials[1])


# speedup vs baseline: 3.3440x; 1.0046x over previous
import functools

import jax
import jax.numpy as jnp
from jax import lax
from jax.experimental import pallas as pl
from jax.experimental.pallas import tpu as pltpu
from jax.experimental.pallas import tpu_sc as plsc


def _ln(x, g, b):
    mu = jnp.mean(x, axis=-1, keepdims=True)
    var = jnp.mean((x - mu) * (x - mu), axis=-1, keepdims=True)
    return (x - mu) / jnp.sqrt(var + 1e-3) * g + b


def _pre_body(q_ref, m_ref, w1q_ref, w1k_ref, b1p_ref,
              w1m_ref, b1m_ref, g1m_ref, be1m_ref,
              w2m_ref, b2m_ref, g2m_ref, be2m_ref,
              w3m_ref, b3m_ref,
              a_ref, b_ref, k_ref):
    q = q_ref[...]
    m = m_ref[...]
    a_ref[...] = jnp.dot(q, w1q_ref[...],
                         preferred_element_type=jnp.float32) + b1p_ref[...]
    b_ref[...] = jnp.dot(m, w1k_ref[...],
                         preferred_element_type=jnp.float32)
    h = jnp.dot(m, w1m_ref[...], preferred_element_type=jnp.float32)
    h = jnp.maximum(_ln(h + b1m_ref[...], g1m_ref[...], be1m_ref[...]), 0.0)
    h = jnp.dot(h, w2m_ref[...], preferred_element_type=jnp.float32)
    h = jnp.maximum(_ln(h + b2m_ref[...], g2m_ref[...], be2m_ref[...]), 0.0)
    k_ref[...] = jnp.dot(h, w3m_ref[...],
                         preferred_element_type=jnp.float32) + b3m_ref[...]


def _precompute(query, memory, w1q, w1k, b1p,
                w1m, b1m, g1m, be1m, w2m, b2m, g2m, be2m, w3m, b3m):
    n, _ = query.shape
    h_dim = w1q.shape[1]
    out_shape = (jax.ShapeDtypeStruct((n, h_dim), jnp.float32),
                 jax.ShapeDtypeStruct((n, h_dim), jnp.float32),
                 jax.ShapeDtypeStruct((n, h_dim), jnp.float32))
    return pl.pallas_call(_pre_body, out_shape=out_shape)(
        query, memory, w1q, w1k, b1p,
        w1m, b1m, g1m, be1m, w2m, b2m, g2m, be2m, w3m, b3m)


def _mlp_body(x_ref, g1_ref, be1_ref, w2_ref, b2_ref, g2_ref, be2_ref,
              w3_ref, b3_ref, h_ref):
    x = x_ref[0]
    h1 = jnp.maximum(_ln(x, g1_ref[...], be1_ref[...]), 0.0)
    h2 = jnp.dot(h1, w2_ref[...], preferred_element_type=jnp.float32) + b2_ref[...]
    h2 = jnp.maximum(_ln(h2, g2_ref[...], be2_ref[...]), 0.0)
    t = jnp.sum(h2 * w3_ref[...], axis=-1) + b3_ref[0, 0]
    h_ref[0, 0] = jnp.tanh(t)


def _edge_mlp(x3, g1p, be1p, w2p, b2p, g2p, be2p, w3row, b3):
    nb, be, h_dim = x3.shape
    wspec = pl.BlockSpec((1, h_dim), lambda i: (0, 0))
    mspec = pl.BlockSpec((h_dim, h_dim), lambda i: (0, 0))
    return pl.pallas_call(
        _mlp_body,
        grid=(nb,),
        in_specs=[pl.BlockSpec((1, be, h_dim), lambda i: (i, 0, 0)),
                  wspec, wspec, mspec, wspec, wspec, wspec, wspec,
                  pl.BlockSpec((1, 1), lambda i: (0, 0))],
        out_specs=pl.BlockSpec((1, 1, be), lambda i: (i, 0, 0)),
        out_shape=jax.ShapeDtypeStruct((nb, 1, be), jnp.float32),
    )(x3, g1p, be1p, w2p, b2p, g2p, be2p, w3row, b3)


def _add_body(a_ref, b_ref, o_ref):
    o_ref[...] = a_ref[...] + b_ref[...]


def _combine(p0, p1):
    return pl.pallas_call(
        _add_body, out_shape=jax.ShapeDtypeStruct(p0.shape, jnp.float32))(p0, p1)


def _make_sc_gather(e_total, h_dim, nc, ns, c):
    nw = nc * ns
    ew = e_total // nw
    nch = ew // c
    mesh = plsc.VectorSubcoreMesh(core_axis_name="c", subcore_axis_name="s")

    @functools.partial(
        pl.kernel, mesh=mesh,
        compiler_params=pltpu.CompilerParams(use_tc_tiling_on_sc=False,
                                             needs_layout_passes=False),
        out_type=jax.ShapeDtypeStruct((e_total, h_dim), jnp.float32),
        scratch_types=[pltpu.VMEM((nch, c), jnp.int32),
                       pltpu.VMEM((nch, c), jnp.int32),
                       pltpu.VMEM((c, h_dim), jnp.float32),
                       pltpu.VMEM((c, h_dim), jnp.float32),
                       pltpu.SemaphoreType.DMA],
    )
    def gk(a_hbm, b_hbm, src_hbm, dst_hbm, x_hbm, sidx, didx, ra, rb, sem):
        wid = lax.axis_index("s") * nc + lax.axis_index("c")
        pltpu.sync_copy(src_hbm.at[wid], sidx)
        pltpu.sync_copy(dst_hbm.at[wid], didx)

        def chunk(ci, carry):
            base = wid * ew + ci * c
            pltpu.async_copy(a_hbm.at[sidx.at[ci]], ra, sem).wait()
            pltpu.async_copy(b_hbm.at[didx.at[ci]], rb, sem).wait()

            def row(i, rcarry):
                for j in range(h_dim // 16):
                    s = pl.ds(j * 16, 16)
                    ra[i, s] = ra[i, s] + rb[i, s]
                return rcarry

            lax.fori_loop(0, c, row, 0)
            pltpu.sync_copy(ra, x_hbm.at[pl.ds(base, c)])
            return carry

        lax.fori_loop(0, nch, chunk, 0)

    return gk


def _make_sc_scatter(n, e_total, h_dim, nc, ns, c):
    nw = nc * ns
    ew = e_total // nw
    nch = ew // c
    mesh = plsc.VectorSubcoreMesh(core_axis_name="c", subcore_axis_name="s")

    @functools.partial(
        pl.kernel, mesh=mesh,
        compiler_params=pltpu.CompilerParams(use_tc_tiling_on_sc=False,
                                             needs_layout_passes=False),
        out_type=jax.ShapeDtypeStruct((nc, n, h_dim), jnp.float32),
        scratch_types=[pltpu.VMEM((nch, c), jnp.int32),
                       pltpu.VMEM((nch, c), jnp.int32),
                       pltpu.VMEM((nch, c), jnp.float32),
                       pltpu.VMEM((2, c, h_dim), jnp.float32),
                       pltpu.VMEM((c,), jnp.int32),
                       pltpu.VMEM((c,), jnp.float32),
                       pltpu.VMEM_SHARED((n, h_dim), jnp.float32),
                       pltpu.SemaphoreType.DMA,
                       pltpu.SemaphoreType.DMA],
    )
    def sk(k_hbm, h_hbm, src_hbm, dst_hbm, z_hbm, out_hbm,
           sidx, didx, hbuf, rows, sv, hv, acc, gsem0, gsem1):
        cid = lax.axis_index("c")
        sid = lax.axis_index("s")
        wid = sid * nc + cid

        @pl.when(sid == 0)
        def _():
            pltpu.sync_copy(z_hbm, acc)

        pltpu.sync_copy(src_hbm.at[wid], sidx)
        pltpu.sync_copy(dst_hbm.at[wid], didx)
        pltpu.sync_copy(h_hbm.at[wid], hbuf)
        plsc.subcore_barrier()
        gsems = (gsem0, gsem1)

        def scale_scatter(ci, b, hdl):
            for j in range(c // 16):
                s = pl.ds(j * 16, 16)
                sv[s] = sidx[ci, s]
                hv[s] = hbuf[ci, s]
            hdl.wait()

            def row(i, rcarry):
                hb = plsc.load_gather(hv, [jnp.full((16,), i, jnp.int32)])
                for j in range(h_dim // 16):
                    s = pl.ds(j * 16, 16)
                    rows[b, i, s] = rows[b, i, s] * hb
                return rcarry

            lax.fori_loop(0, c, row, 0)
            pltpu.sync_copy(rows.at[b], acc.at[sv], add=True)

        def chunk(ci, carry):
            hdl = pltpu.async_copy(k_hbm.at[didx.at[ci]], rows.at[0],
                                   gsems[0])
            scale_scatter(ci, 0, hdl)
            return carry

        lax.fori_loop(0, nch, chunk, 0)
        plsc.subcore_barrier()

        @pl.when(sid == 0)
        def _():
            pltpu.sync_copy(acc, out_hbm.at[cid])

    return sk


def kernel(query, memory, adj_indices, W1p, b1p, g1p, be1p, W2p, b2p, g2p,
           be2p, W3p, b3p, W1m, b1m, g1m, be1m, W2m, b2m, g2m, be2m, W3m,
           b3m):
    n, d = query.shape
    e_total = adj_indices.shape[0]
    h_dim = W1p.shape[1]

    info = plsc.get_sparse_core_info()
    nc, ns = info.num_cores, info.num_subcores
    c = 80

    nw = nc * ns
    nch = e_total // (nw * c)
    src = adj_indices[:, 0]
    dst = adj_indices[:, 1]
    src3 = src.reshape(nw, nch, c)
    dst3 = dst.reshape(nw, nch, c)

    a_tab, b_tab, k_tab = _precompute(
        query, memory, W1p[:d], W1p[d:], b1p.reshape(1, h_dim),
        W1m, b1m.reshape(1, h_dim), g1m.reshape(1, h_dim),
        be1m.reshape(1, h_dim), W2m, b2m.reshape(1, h_dim),
        g2m.reshape(1, h_dim), be2m.reshape(1, h_dim), W3m,
        b3m.reshape(1, h_dim))

    x = _make_sc_gather(e_total, h_dim, nc, ns, c)(a_tab, b_tab, src3, dst3)

    be = 2560
    nb = e_total // be
    h3 = _edge_mlp(x.reshape(nb, be, h_dim),
                   g1p.reshape(1, h_dim), be1p.reshape(1, h_dim),
                   W2p, b2p.reshape(1, h_dim),
                   g2p.reshape(1, h_dim), be2p.reshape(1, h_dim),
                   W3p.reshape(1, h_dim), b3p.reshape(1, 1))
    h_edges = h3.reshape(nw, nch, c)

    zeros = jnp.zeros((n, h_dim), jnp.float32)
    partials = _make_sc_scatter(n, e_total, h_dim, nc, ns, c)(
        k_tab, h_edges, src3, dst3, zeros)

    return _combine(partials[0], partials[1])


# speedup vs baseline: 3.6140x; 1.0807x over previous
import functools

import jax
import jax.numpy as jnp
from jax import lax
from jax.experimental import pallas as pl
from jax.experimental.pallas import tpu as pltpu
from jax.experimental.pallas import tpu_sc as plsc


def _ln(x, g, b):
    mu = jnp.mean(x, axis=-1, keepdims=True)
    var = jnp.mean((x - mu) * (x - mu), axis=-1, keepdims=True)
    return (x - mu) / jnp.sqrt(var + 1e-3) * g + b


def _pre_body(q_ref, m_ref, w1q_ref, w1k_ref, b1p_ref,
              w1m_ref, b1m_ref, g1m_ref, be1m_ref,
              w2m_ref, b2m_ref, g2m_ref, be2m_ref,
              w3m_ref, b3m_ref,
              a_ref, b_ref, k_ref):
    q = q_ref[...]
    m = m_ref[...]
    a_ref[...] = jnp.dot(q, w1q_ref[...],
                         preferred_element_type=jnp.float32) + b1p_ref[...]
    b_ref[...] = jnp.dot(m, w1k_ref[...],
                         preferred_element_type=jnp.float32)
    h = jnp.dot(m, w1m_ref[...], preferred_element_type=jnp.float32)
    h = jnp.maximum(_ln(h + b1m_ref[...], g1m_ref[...], be1m_ref[...]), 0.0)
    h = jnp.dot(h, w2m_ref[...], preferred_element_type=jnp.float32)
    h = jnp.maximum(_ln(h + b2m_ref[...], g2m_ref[...], be2m_ref[...]), 0.0)
    k_ref[...] = jnp.dot(h, w3m_ref[...],
                         preferred_element_type=jnp.float32) + b3m_ref[...]


def _precompute(query, memory, w1q, w1k, b1p,
                w1m, b1m, g1m, be1m, w2m, b2m, g2m, be2m, w3m, b3m):
    n, _ = query.shape
    h_dim = w1q.shape[1]
    out_shape = (jax.ShapeDtypeStruct((n, h_dim), jnp.float32),
                 jax.ShapeDtypeStruct((n, h_dim), jnp.float32),
                 jax.ShapeDtypeStruct((n, h_dim), jnp.float32))
    return pl.pallas_call(_pre_body, out_shape=out_shape)(
        query, memory, w1q, w1k, b1p,
        w1m, b1m, g1m, be1m, w2m, b2m, g2m, be2m, w3m, b3m)


def _mlp_body(x_ref, g1_ref, be1_ref, w2_ref, b2_ref, g2_ref, be2_ref,
              w3_ref, b3_ref, h_ref):
    x = x_ref[0]
    h1 = jnp.maximum(_ln(x, g1_ref[...], be1_ref[...]), 0.0)
    h2 = jnp.dot(h1, w2_ref[...], preferred_element_type=jnp.float32) + b2_ref[...]
    h2 = jnp.maximum(_ln(h2, g2_ref[...], be2_ref[...]), 0.0)
    t = jnp.sum(h2 * w3_ref[...], axis=-1) + b3_ref[0, 0]
    h_ref[0, 0] = jnp.tanh(t)


def _edge_mlp(x3, g1p, be1p, w2p, b2p, g2p, be2p, w3row, b3):
    nb, be, h_dim = x3.shape
    wspec = pl.BlockSpec((1, h_dim), lambda i: (0, 0))
    mspec = pl.BlockSpec((h_dim, h_dim), lambda i: (0, 0))
    return pl.pallas_call(
        _mlp_body,
        grid=(nb,),
        in_specs=[pl.BlockSpec((1, be, h_dim), lambda i: (i, 0, 0)),
                  wspec, wspec, mspec, wspec, wspec, wspec, wspec,
                  pl.BlockSpec((1, 1), lambda i: (0, 0))],
        out_specs=pl.BlockSpec((1, 1, be), lambda i: (i, 0, 0)),
        out_shape=jax.ShapeDtypeStruct((nb, 1, be), jnp.float32),
    )(x3, g1p, be1p, w2p, b2p, g2p, be2p, w3row, b3)


def _add_body(a_ref, b_ref, o_ref):
    o_ref[...] = a_ref[...] + b_ref[...]


def _combine(p0, p1):
    return pl.pallas_call(
        _add_body, out_shape=jax.ShapeDtypeStruct(p0.shape, jnp.float32))(p0, p1)


def _make_sc_gather(e_total, h_dim, nc, ns, c):
    nw = nc * ns
    ew = e_total // nw
    nch = ew // c
    mesh = plsc.VectorSubcoreMesh(core_axis_name="c", subcore_axis_name="s")

    @functools.partial(
        pl.kernel, mesh=mesh,
        compiler_params=pltpu.CompilerParams(use_tc_tiling_on_sc=False,
                                             needs_layout_passes=False),
        out_type=jax.ShapeDtypeStruct((e_total, h_dim), jnp.float32),
        scratch_types=[pltpu.VMEM((nch, c), jnp.int32),
                       pltpu.VMEM((nch, c), jnp.int32),
                       pltpu.VMEM((c, h_dim), jnp.float32),
                       pltpu.VMEM((c, h_dim), jnp.float32),
                       pltpu.SemaphoreType.DMA,
                       pltpu.SemaphoreType.DMA],
    )
    def gk(a_hbm, b_hbm, src_hbm, dst_hbm, x_hbm, sidx, didx, ra, rb,
           sema, semb):
        wid = lax.axis_index("s") * nc + lax.axis_index("c")
        pltpu.sync_copy(src_hbm.at[wid], sidx)
        pltpu.sync_copy(dst_hbm.at[wid], didx)

        def chunk(ci, carry):
            base = wid * ew + ci * c
            hdla = pltpu.async_copy(a_hbm.at[sidx.at[ci]], ra, sema)
            hdlb = pltpu.async_copy(b_hbm.at[didx.at[ci]], rb, semb)
            hdla.wait()
            hdlb.wait()

            def row(i, rcarry):
                for j in range(h_dim // 16):
                    s = pl.ds(j * 16, 16)
                    ra[i, s] = ra[i, s] + rb[i, s]
                return rcarry

            lax.fori_loop(0, c, row, 0)
            pltpu.sync_copy(ra, x_hbm.at[pl.ds(base, c)])
            return carry

        lax.fori_loop(0, nch, chunk, 0)

    return gk


def _make_sc_scatter(n, e_total, h_dim, nc, ns, c):
    nw = nc * ns
    ew = e_total // nw
    nch = ew // c
    mesh = plsc.VectorSubcoreMesh(core_axis_name="c", subcore_axis_name="s")

    @functools.partial(
        pl.kernel, mesh=mesh,
        compiler_params=pltpu.CompilerParams(use_tc_tiling_on_sc=False,
                                             needs_layout_passes=False),
        out_type=jax.ShapeDtypeStruct((nc, n, h_dim), jnp.float32),
        scratch_types=[pltpu.VMEM((nch, c), jnp.int32),
                       pltpu.VMEM((nch, c), jnp.int32),
                       pltpu.VMEM((nch, c), jnp.float32),
                       pltpu.VMEM((2, c, h_dim), jnp.float32),
                       pltpu.VMEM((c,), jnp.int32),
                       pltpu.VMEM((c,), jnp.float32),
                       pltpu.VMEM_SHARED((n, h_dim), jnp.float32),
                       pltpu.SemaphoreType.DMA,
                       pltpu.SemaphoreType.DMA],
    )
    def sk(k_hbm, h_hbm, src_hbm, dst_hbm, z_hbm, out_hbm,
           sidx, didx, hbuf, rows, sv, hv, acc, gsem0, gsem1):
        cid = lax.axis_index("c")
        sid = lax.axis_index("s")
        wid = sid * nc + cid

        @pl.when(sid == 0)
        def _():
            pltpu.sync_copy(z_hbm, acc)

        pltpu.sync_copy(src_hbm.at[wid], sidx)
        pltpu.sync_copy(dst_hbm.at[wid], didx)
        pltpu.sync_copy(h_hbm.at[wid], hbuf)
        plsc.subcore_barrier()
        gsems = (gsem0, gsem1)

        def scale_scatter(ci, b, hdl):
            for j in range(c // 16):
                s = pl.ds(j * 16, 16)
                sv[s] = sidx[ci, s]
                hv[s] = hbuf[ci, s]
            hdl.wait()

            def row(i, rcarry):
                hb = plsc.load_gather(hv, [jnp.full((16,), i, jnp.int32)])
                for j in range(h_dim // 16):
                    s = pl.ds(j * 16, 16)
                    rows[b, i, s] = rows[b, i, s] * hb
                return rcarry

            lax.fori_loop(0, c, row, 0)
            pltpu.sync_copy(rows.at[b], acc.at[sv], add=True)

        def chunk(ci, carry):
            hdl = pltpu.async_copy(k_hbm.at[didx.at[ci]], rows.at[0],
                                   gsems[0])
            scale_scatter(ci, 0, hdl)
            return carry

        lax.fori_loop(0, nch, chunk, 0)
        plsc.subcore_barrier()

        @pl.when(sid == 0)
        def _():
            pltpu.sync_copy(acc, out_hbm.at[cid])

    return sk


def kernel(query, memory, adj_indices, W1p, b1p, g1p, be1p, W2p, b2p, g2p,
           be2p, W3p, b3p, W1m, b1m, g1m, be1m, W2m, b2m, g2m, be2m, W3m,
           b3m):
    n, d = query.shape
    e_total = adj_indices.shape[0]
    h_dim = W1p.shape[1]

    info = plsc.get_sparse_core_info()
    nc, ns = info.num_cores, info.num_subcores
    c = 80

    nw = nc * ns
    nch = e_total // (nw * c)
    src = adj_indices[:, 0]
    dst = adj_indices[:, 1]
    src3 = src.reshape(nw, nch, c)
    dst3 = dst.reshape(nw, nch, c)

    a_tab, b_tab, k_tab = _precompute(
        query, memory, W1p[:d], W1p[d:], b1p.reshape(1, h_dim),
        W1m, b1m.reshape(1, h_dim), g1m.reshape(1, h_dim),
        be1m.reshape(1, h_dim), W2m, b2m.reshape(1, h_dim),
        g2m.reshape(1, h_dim), be2m.reshape(1, h_dim), W3m,
        b3m.reshape(1, h_dim))

    x = _make_sc_gather(e_total, h_dim, nc, ns, c)(a_tab, b_tab, src3, dst3)

    be = 2560
    nb = e_total // be
    h3 = _edge_mlp(x.reshape(nb, be, h_dim),
                   g1p.reshape(1, h_dim), be1p.reshape(1, h_dim),
                   W2p, b2p.reshape(1, h_dim),
                   g2p.reshape(1, h_dim), be2p.reshape(1, h_dim),
                   W3p.reshape(1, h_dim), b3p.reshape(1, 1))
    h_edges = h3.reshape(nw, nch, c)

    zeros = jnp.zeros((n, h_dim), jnp.float32)
    partials = _make_sc_scatter(n, e_total, h_dim, nc, ns, c)(
        k_tab, h_edges, src3, dst3, zeros)

    return _combine(partials[0], partials[1])
